# baseline (speedup 1.0000x reference)
import functools

import jax
import jax.numpy as jnp
from jax import lax
from jax.experimental import pallas as pl
from jax.experimental.pallas import tpu as pltpu
from jax.experimental.pallas import tpu_sc as plsc

HEAD_HALF = 64


def kernel(position_ids, cos_cache, sin_cache):
    bsz, seq = position_ids.shape
    total = bsz * seq
    info = plsc.get_sparse_core_info()
    nw = info.num_cores * info.num_subcores
    b_per_w = total // nw

    idx_flat = position_ids.reshape(total)
    mesh = plsc.VectorSubcoreMesh(core_axis_name="c", subcore_axis_name="s")

    @functools.partial(
        pl.kernel,
        mesh=mesh,
        out_type=(
            jax.ShapeDtypeStruct((total, HEAD_HALF), jnp.float32),
            jax.ShapeDtypeStruct((total, HEAD_HALF), jnp.float32),
        ),
        scratch_types=[
            pltpu.VMEM((b_per_w,), jnp.int32),
            pltpu.VMEM((b_per_w, HEAD_HALF), jnp.float32),
            pltpu.SemaphoreType.DMA,
        ],
        compiler_params=pltpu.CompilerParams(use_tc_tiling_on_sc=False),
    )
    def rope_gather(idx_hbm, cos_hbm, sin_hbm, cos_out, sin_out, idx_v, rows_v, sem):
        wid = lax.axis_index("s") * info.num_cores + lax.axis_index("c")
        base = wid * b_per_w
        pltpu.sync_copy(idx_hbm.at[pl.ds(base, b_per_w)], idx_v)
        pltpu.async_copy(cos_hbm.at[idx_v], rows_v, sem).wait()
        pltpu.sync_copy(rows_v, cos_out.at[pl.ds(base, b_per_w)])
        pltpu.async_copy(sin_hbm.at[idx_v], rows_v, sem).wait()
        pltpu.sync_copy(rows_v, sin_out.at[pl.ds(base, b_per_w)])

    cos_flat, sin_flat = rope_gather(idx_flat, cos_cache, sin_cache)
    return (
        cos_flat.reshape(bsz, seq, HEAD_HALF),
        sin_flat.reshape(bsz, seq, HEAD_HALF),
    )


# speedup vs baseline: 1.0024x; 1.0024x over previous
import functools

import jax
import jax.numpy as jnp
from jax import lax
from jax.experimental import pallas as pl
from jax.experimental.pallas import tpu as pltpu
from jax.experimental.pallas import tpu_sc as plsc

HEAD_HALF = 64
CHUNK = 256


def kernel(position_ids, cos_cache, sin_cache):
    bsz, seq = position_ids.shape
    total = bsz * seq
    info = plsc.get_sparse_core_info()
    nw = info.num_cores * info.num_subcores
    b_per_w = total // nw
    nch = b_per_w // CHUNK

    idx_flat = position_ids.reshape(total)
    mesh = plsc.VectorSubcoreMesh(core_axis_name="c", subcore_axis_name="s")

    @functools.partial(
        pl.kernel,
        mesh=mesh,
        out_type=(
            jax.ShapeDtypeStruct((total, HEAD_HALF), jnp.float32),
            jax.ShapeDtypeStruct((total, HEAD_HALF), jnp.float32),
        ),
        scratch_types=[
            pltpu.VMEM((b_per_w,), jnp.int32),
            pltpu.VMEM((2, CHUNK, HEAD_HALF), jnp.float32),
            pltpu.VMEM((2, CHUNK, HEAD_HALF), jnp.float32),
            pltpu.SemaphoreType.DMA,
            pltpu.SemaphoreType.DMA,
        ],
        compiler_params=pltpu.CompilerParams(use_tc_tiling_on_sc=False),
    )
    def rope_gather(idx_hbm, cos_hbm, sin_hbm, cos_out, sin_out,
                    idx_v, cos_v, sin_v, gsem, wsem):
        wid = lax.axis_index("s") * info.num_cores + lax.axis_index("c")
        base = wid * b_per_w
        pltpu.sync_copy(idx_hbm.at[pl.ds(base, b_per_w)], idx_v)

        gathers = [None] * nch
        writes = [None] * nch

        def start_gather(i):
            slot = i % 2
            off = i * CHUNK
            c = pltpu.make_async_copy(
                cos_hbm.at[idx_v.at[pl.ds(off, CHUNK)]], cos_v.at[slot], gsem)
            s = pltpu.make_async_copy(
                sin_hbm.at[idx_v.at[pl.ds(off, CHUNK)]], sin_v.at[slot], gsem)
            c.start()
            s.start()
            gathers[i] = (c, s)

        def start_writes(i):
            slot = i % 2
            off = base + i * CHUNK
            c = pltpu.make_async_copy(
                cos_v.at[slot], cos_out.at[pl.ds(off, CHUNK)], wsem)
            s = pltpu.make_async_copy(
                sin_v.at[slot], sin_out.at[pl.ds(off, CHUNK)], wsem)
            c.start()
            s.start()
            writes[i] = (c, s)

        def wait_pair(pair):
            pair[0].wait()
            pair[1].wait()

        start_gather(0)
        for i in range(nch):
            if i + 1 < nch:
                if i >= 1:
                    wait_pair(writes[i - 1])
                start_gather(i + 1)
            wait_pair(gathers[i])
            start_writes(i)
        for i in range(max(nch - 2, 0), nch):
            wait_pair(writes[i])

    cos_flat, sin_flat = rope_gather(idx_flat, cos_cache, sin_cache)
    return (
        cos_flat.reshape(bsz, seq, HEAD_HALF),
        sin_flat.reshape(bsz, seq, HEAD_HALF),
    )


# speedup vs baseline: 2.2952x; 2.2897x over previous
import functools

import jax
import jax.numpy as jnp
from jax import lax
from jax.experimental import pallas as pl
from jax.experimental.pallas import tpu as pltpu
from jax.experimental.pallas import tpu_sc as plsc

LANES = 16


def kernel(position_ids, cos_cache, sin_cache):
    bsz, seq = position_ids.shape
    total = bsz * seq
    n_pos, head_half = cos_cache.shape
    info = plsc.get_sparse_core_info()
    nw = info.num_cores * info.num_subcores
    h_per_w = head_half // nw

    idx_flat = position_ids.reshape(total)
    cos_t = cos_cache.T
    sin_t = sin_cache.T

    mesh = plsc.VectorSubcoreMesh(core_axis_name="c", subcore_axis_name="s")

    @functools.partial(
        pl.kernel,
        mesh=mesh,
        out_type=(
            jax.ShapeDtypeStruct((bsz, head_half, seq), jnp.float32),
            jax.ShapeDtypeStruct((bsz, head_half, seq), jnp.float32),
        ),
        scratch_types=[
            pltpu.VMEM((total,), jnp.int32),
            pltpu.VMEM((n_pos,), jnp.float32),
            pltpu.VMEM((n_pos,), jnp.float32),
            pltpu.VMEM((seq,), jnp.float32),
            pltpu.VMEM((seq,), jnp.float32),
        ],
        compiler_params=pltpu.CompilerParams(needs_layout_passes=False),
    )
    def rope_gather(idx_hbm, cos_hbm, sin_hbm, cos_out, sin_out,
                    idx_v, row_cos, row_sin, out_cos_v, out_sin_v):
        wid = lax.axis_index("s") * info.num_cores + lax.axis_index("c")
        h0 = wid * h_per_w
        pltpu.sync_copy(idx_hbm, idx_v)
        for j in range(h_per_w):
            h = h0 + j
            pltpu.sync_copy(cos_hbm.at[h], row_cos)
            pltpu.sync_copy(sin_hbm.at[h], row_sin)
            for b in range(bsz):
                base = b * seq

                def body(s):
                    off = s * LANES
                    iv = idx_v[pl.ds(base + off, LANES)]
                    out_cos_v[pl.ds(off, LANES)] = plsc.load_gather(
                        row_cos, [iv])
                    out_sin_v[pl.ds(off, LANES)] = plsc.load_gather(
                        row_sin, [iv])

                plsc.parallel_loop(0, seq // LANES, 1, unroll=8)(body)
                pltpu.sync_copy(out_cos_v, cos_out.at[b, h])
                pltpu.sync_copy(out_sin_v, sin_out.at[b, h])

    cos_r, sin_r = rope_gather(idx_flat, cos_t, sin_t)
    return (
        jnp.transpose(cos_r, (0, 2, 1)),
        jnp.transpose(sin_r, (0, 2, 1)),
    )


# speedup vs baseline: 2.6732x; 1.1647x over previous
import functools

import jax
import jax.numpy as jnp
from jax import lax
from jax.experimental import pallas as pl
from jax.experimental.pallas import tpu as pltpu
from jax.experimental.pallas import tpu_sc as plsc

LANES = 16


def kernel(position_ids, cos_cache, sin_cache):
    bsz, seq = position_ids.shape
    total = bsz * seq
    n_pos, head_half = cos_cache.shape
    info = plsc.get_sparse_core_info()
    ns = info.num_subcores
    h_per_w = head_half // ns

    idx_flat = position_ids.reshape(total)
    cos_t = cos_cache.T
    sin_t = sin_cache.T

    mesh = plsc.VectorSubcoreMesh(core_axis_name="c", subcore_axis_name="s")

    @functools.partial(
        pl.kernel,
        mesh=mesh,
        out_type=(
            jax.ShapeDtypeStruct((bsz, head_half, seq), jnp.float32),
            jax.ShapeDtypeStruct((bsz, head_half, seq), jnp.float32),
        ),
        scratch_types=[
            pltpu.VMEM((total,), jnp.int32),
            pltpu.VMEM((n_pos,), jnp.float32),
            pltpu.VMEM((n_pos,), jnp.float32),
            pltpu.VMEM((seq,), jnp.float32),
            pltpu.VMEM((seq,), jnp.float32),
            pltpu.SemaphoreType.DMA,
            pltpu.SemaphoreType.DMA,
            pltpu.SemaphoreType.DMA,
            pltpu.SemaphoreType.DMA,
        ],
        compiler_params=pltpu.CompilerParams(needs_layout_passes=False),
    )
    def rope_gather(idx_hbm, cos_hbm, sin_hbm, cos_out, sin_out,
                    idx_v, row0, row1, ob0, ob1, isem, rsem0, rsem1, osem):
        core = lax.axis_index("c")
        tile = lax.axis_index("s")
        h0 = tile * h_per_w
        rows = (row0, row1)
        rsems = (rsem0, rsem1)
        obufs = (ob0, ob1)

        def process(tab, outp):
            idx_cp = pltpu.make_async_copy(idx_hbm, idx_v, isem)
            idx_cp.start()
            row_cps = [None] * h_per_w
            out_cps = [None] * (h_per_w * bsz)

            def start_row(j):
                cp = pltpu.make_async_copy(
                    tab.at[h0 + j], rows[j % 2], rsems[j % 2])
                cp.start()
                row_cps[j] = cp

            start_row(0)
            idx_cp.wait()
            for j in range(h_per_w):
                if j + 1 < h_per_w:
                    start_row(j + 1)
                row_cps[j].wait()
                row = rows[j % 2]
                for b in range(bsz):
                    t = j * bsz + b
                    slot = t % 2
                    if t >= 2:
                        out_cps[t - 2].wait()
                    base = b * seq
                    obs = obufs[slot]

                    def body(s):
                        off = s * LANES
                        iv = idx_v[pl.ds(base + off, LANES)]
                        obs[pl.ds(off, LANES)] = plsc.load_gather(row, [iv])

                    plsc.parallel_loop(0, seq // LANES, 1, unroll=8)(body)
                    cp = pltpu.make_async_copy(
                        obs, outp.at[b, h0 + j], osem)
                    cp.start()
                    out_cps[t] = cp
            out_cps[-2].wait()
            out_cps[-1].wait()

        @pl.when(core == 0)
        def _():
            process(cos_hbm, cos_out)

        @pl.when(core == 1)
        def _():
            process(sin_hbm, sin_out)

    cos_r, sin_r = rope_gather(idx_flat, cos_t, sin_t)
    return (
        jnp.transpose(cos_r, (0, 2, 1)),
        jnp.transpose(sin_r, (0, 2, 1)),
    )


# speedup vs baseline: 2.9179x; 1.0915x over previous
import functools

import jax
import jax.numpy as jnp
from jax import lax
from jax.experimental import pallas as pl
from jax.experimental.pallas import tpu as pltpu
from jax.experimental.pallas import tpu_sc as plsc

LANES = 16


def kernel(position_ids, cos_cache, sin_cache):
    bsz, seq = position_ids.shape
    total = bsz * seq
    n_pos, head_half = cos_cache.shape
    info = plsc.get_sparse_core_info()
    ns = info.num_subcores
    h_per_w = head_half // ns

    idx_flat = position_ids.reshape(total)
    cos_t = cos_cache.T
    sin_t = sin_cache.T

    mesh = plsc.VectorSubcoreMesh(core_axis_name="c", subcore_axis_name="s")

    @functools.partial(
        pl.kernel,
        mesh=mesh,
        out_type=(
            jax.ShapeDtypeStruct((bsz, head_half, seq), jnp.float32),
            jax.ShapeDtypeStruct((bsz, head_half, seq), jnp.float32),
        ),
        scratch_types=[
            pltpu.VMEM_SHARED((total,), jnp.int32),
            pltpu.VMEM((total,), jnp.int32),
            pltpu.VMEM((n_pos,), jnp.float32),
            pltpu.VMEM((n_pos,), jnp.float32),
            pltpu.VMEM((seq,), jnp.float32),
            pltpu.VMEM((seq,), jnp.float32),
            pltpu.SemaphoreType.DMA,
            pltpu.SemaphoreType.DMA,
            pltpu.SemaphoreType.DMA,
            pltpu.SemaphoreType.DMA,
        ],
        compiler_params=pltpu.CompilerParams(needs_layout_passes=False),
    )
    def rope_gather(idx_hbm, cos_hbm, sin_hbm, cos_out, sin_out,
                    idx_sh, idx_v, row0, row1, ob0, ob1,
                    isem, rsem0, rsem1, osem):
        core = lax.axis_index("c")
        tile = lax.axis_index("s")
        h0 = tile * h_per_w
        rows = (row0, row1)
        rsems = (rsem0, rsem1)
        obufs = (ob0, ob1)

        def process(tab, outp):
            row_cps = [None] * h_per_w
            out_cps = [None] * (h_per_w * bsz)

            def start_row(j):
                cp = pltpu.make_async_copy(
                    tab.at[h0 + j], rows[j % 2], rsems[j % 2])
                cp.start()
                row_cps[j] = cp

            start_row(0)
            @pl.when(tile == 0)
            def _():
                pltpu.sync_copy(idx_hbm, idx_sh)

            plsc.subcore_barrier()
            idx_cp = pltpu.make_async_copy(idx_sh, idx_v, isem)
            idx_cp.start()
            idx_cp.wait()
            for j in range(h_per_w):
                if j + 1 < h_per_w:
                    start_row(j + 1)
                row_cps[j].wait()
                row = rows[j % 2]
                for b in range(bsz):
                    t = j * bsz + b
                    slot = t % 2
                    if t >= 2:
                        out_cps[t - 2].wait()
                    base = b * seq
                    obs = obufs[slot]

                    def body(s):
                        off = s * LANES
                        iv = idx_v[pl.ds(base + off, LANES)]
                        obs[pl.ds(off, LANES)] = plsc.load_gather(row, [iv])

                    plsc.parallel_loop(0, seq // LANES, 1, unroll=8)(body)
                    cp = pltpu.make_async_copy(
                        obs, outp.at[b, h0 + j], osem)
                    cp.start()
                    out_cps[t] = cp
            out_cps[-2].wait()
            out_cps[-1].wait()

        @pl.when(core == 0)
        def _():
            process(cos_hbm, cos_out)

        @pl.when(core == 1)
        def _():
            process(sin_hbm, sin_out)

    cos_r, sin_r = rope_gather(idx_flat, cos_t, sin_t)
    return (
        jnp.transpose(cos_r, (0, 2, 1)),
        jnp.transpose(sin_r, (0, 2, 1)),
    )
